# baseline (speedup 1.0000x reference)
import jax
import jax.numpy as jnp
import numpy as np
from jax.experimental import pallas as pl
from jax.experimental.pallas import tpu as pltpu

_NUM_LEARNERS = 1000
_NUM_TOPICS = 128
_NUM_KC = 256
_NUM_TIMES = 10
_DIM = 5
_SIGMA_V = 0.1
_SIGMA2_R = 0.1

_ROWS = _NUM_TIMES * _NUM_LEARNERS
_KD = _NUM_KC * _DIM
_BLK = 1000
_GRID = _ROWS // _BLK

_LOG_CONST = np.float32(-np.log(_SIGMA2_R) - 0.5 * np.log(2.0 * np.pi))
_INV_SIGMA = np.float32(1.0 / _SIGMA2_R)


def _kl_kernel(nw_ref, v_ref, noise_ref, w_ref, b_ref,
               u_ref, r_ref, c_ref, out_ref, a_scr, crow_scr):
    step = pl.program_id(0)

    @pl.when(step == 0)
    def _init():
        nw = nw_ref[...]
        ones_col = jnp.ones((1, _NUM_TOPICS), dtype=jnp.float32)
        colsum = jax.lax.dot_general(
            nw, ones_col, (((0,), (1,)), ((), ())),
            preferred_element_type=jnp.float32)
        vm = colsum * v_ref[...] + noise_ref[...]

        row = jax.lax.broadcasted_iota(jnp.int32, (_KD, _NUM_KC), 0)
        col = jax.lax.broadcasted_iota(jnp.int32, (_KD, _NUM_KC), 1)
        t = row - _DIM * col
        e = jnp.zeros((_KD, _NUM_KC), dtype=jnp.float32)
        for d in range(_DIM):
            e = jnp.where(t == d, w_ref[0, d], e)
        a_scr[...] = jax.lax.dot_general(
            e, vm, (((1,), (1,)), ((), ())),
            preferred_element_type=jnp.float32)
        ones_kc = jnp.ones((1, _NUM_KC), dtype=jnp.float32)
        crow_scr[...] = b_ref[0, 0] * jax.lax.dot_general(
            ones_kc, vm, (((1,), (1,)), ((), ())),
            preferred_element_type=jnp.float32)
        out_ref[0, 0] = jnp.float32(0.0)

    mu_pre = jax.lax.dot_general(
        u_ref[...], a_scr[...], (((1,), (0,)), ((), ())),
        preferred_element_type=jnp.float32) + crow_scr[...]
    mu = jax.nn.sigmoid(mu_pre)
    z = (r_ref[...] - mu) * _INV_SIGMA
    lp = _LOG_CONST - 0.5 * (z * z)
    masked = jnp.where(c_ref[...] == 1, lp, jnp.float32(0.0))
    out_ref[0, 0] += jnp.sum(masked)


def kernel(Q, V, R, C, neighbor_weights, U, W_emb, b_emb):
    del Q
    u_flat = U.reshape(_ROWS, _KD)
    r_flat = R.reshape(_ROWS, _NUM_TOPICS)
    c_flat = C.reshape(_ROWS, _NUM_TOPICS)
    noise = (jax.random.normal(jax.random.key(42), (_NUM_KC,), dtype=jnp.float32)
             * jnp.float32(_SIGMA_V)).reshape(1, _NUM_KC)
    b2 = b_emb.reshape(1, 1)

    full = lambda shape: pl.BlockSpec(shape, lambda i: (0, 0))
    out = pl.pallas_call(
        _kl_kernel,
        grid=(_GRID,),
        in_specs=[
            full((_NUM_TOPICS, _NUM_TOPICS)),
            full((_NUM_TOPICS, _NUM_KC)),
            full((1, _NUM_KC)),
            pl.BlockSpec(memory_space=pltpu.MemorySpace.SMEM),
            pl.BlockSpec(memory_space=pltpu.MemorySpace.SMEM),
            pl.BlockSpec((_BLK, _KD), lambda i: (i, 0)),
            pl.BlockSpec((_BLK, _NUM_TOPICS), lambda i: (i, 0)),
            pl.BlockSpec((_BLK, _NUM_TOPICS), lambda i: (i, 0)),
        ],
        out_specs=pl.BlockSpec(memory_space=pltpu.MemorySpace.SMEM),
        out_shape=jax.ShapeDtypeStruct((1, 1), jnp.float32),
        scratch_shapes=[
            pltpu.VMEM((_KD, _NUM_TOPICS), jnp.float32),
            pltpu.VMEM((1, _NUM_TOPICS), jnp.float32),
        ],
    )(neighbor_weights, V, noise, W_emb, b2, u_flat, r_flat, c_flat)
    return out[0, 0]


# speedup vs baseline: 8.3722x; 8.3722x over previous
import jax
import jax.numpy as jnp
import numpy as np
from jax.experimental import pallas as pl
from jax.experimental.pallas import tpu as pltpu

_NUM_LEARNERS = 1000
_NUM_TOPICS = 128
_NUM_KC = 256
_NUM_TIMES = 10
_DIM = 5
_SIGMA_V = 0.1
_SIGMA2_R = 0.1

_LOG_CONST = np.float32(-np.log(_SIGMA2_R) - 0.5 * np.log(2.0 * np.pi))
_INV_SIGMA = np.float32(1.0 / _SIGMA2_R)


def _kl_kernel(nw_ref, v_ref, noise_ref, w_ref, b_ref,
               u_ref, r_ref, c_ref, out_ref, vm_scr):
    step = pl.program_id(0)

    @pl.when(step == 0)
    def _init():
        nw = nw_ref[...]
        ones_col = jnp.ones((1, _NUM_TOPICS), dtype=jnp.float32)
        colsum = jax.lax.dot_general(
            nw, ones_col, (((0,), (1,)), ((), ())),
            preferred_element_type=jnp.float32)
        vm_scr[...] = colsum * v_ref[...] + noise_ref[...]
        out_ref[0, 0] = jnp.float32(0.0)

    ue = u_ref[0, 0] * w_ref[0, 0] + b_ref[0, 0]
    for d in range(1, _DIM):
        ue = ue + u_ref[0, d] * w_ref[0, d]
    mu_pre = jax.lax.dot_general(
        ue, vm_scr[...], (((1,), (1,)), ((), ())),
        preferred_element_type=jnp.float32)
    mu = jax.nn.sigmoid(mu_pre)
    z = (r_ref[0] - mu) * _INV_SIGMA
    lp = _LOG_CONST - 0.5 * (z * z)
    masked = jnp.where(c_ref[0] == 1, lp, jnp.float32(0.0))
    out_ref[0, 0] += jnp.sum(masked)


def kernel(Q, V, R, C, neighbor_weights, U, W_emb, b_emb):
    del Q
    u_t = jnp.transpose(U, (0, 3, 1, 2))
    noise = (jax.random.normal(jax.random.key(42), (_NUM_KC,), dtype=jnp.float32)
             * jnp.float32(_SIGMA_V)).reshape(1, _NUM_KC)
    b2 = b_emb.reshape(1, 1)

    full = lambda shape: pl.BlockSpec(shape, lambda i: (0,) * len(shape))
    out = pl.pallas_call(
        _kl_kernel,
        grid=(_NUM_TIMES,),
        in_specs=[
            full((_NUM_TOPICS, _NUM_TOPICS)),
            full((_NUM_TOPICS, _NUM_KC)),
            full((1, _NUM_KC)),
            pl.BlockSpec(memory_space=pltpu.MemorySpace.SMEM),
            pl.BlockSpec(memory_space=pltpu.MemorySpace.SMEM),
            pl.BlockSpec((1, _DIM, _NUM_LEARNERS, _NUM_KC),
                         lambda i: (i, 0, 0, 0)),
            pl.BlockSpec((1, _NUM_LEARNERS, _NUM_TOPICS),
                         lambda i: (i, 0, 0)),
            pl.BlockSpec((1, _NUM_LEARNERS, _NUM_TOPICS),
                         lambda i: (i, 0, 0)),
        ],
        out_specs=pl.BlockSpec(memory_space=pltpu.MemorySpace.SMEM),
        out_shape=jax.ShapeDtypeStruct((1, 1), jnp.float32),
        scratch_shapes=[
            pltpu.VMEM((_NUM_TOPICS, _NUM_KC), jnp.float32),
        ],
    )(neighbor_weights, V, noise, W_emb, b2, u_t, R, C)
    return out[0, 0]
